# baseline (speedup 1.0000x reference)
import functools

import jax
import jax.numpy as jnp
from jax.experimental import pallas as pl

N = 10000
D = 128
H = 128
G = 64
EPS = 1e-5

TM1 = 200
TM2 = 1000


def _pass1_kernel(x_ref, adj_ref, w1_ref, b1_ref, z_ref, s_ref, ss_ref):
    pooled = jnp.dot(adj_ref[...], x_ref[...], preferred_element_type=jnp.float32)
    z = jnp.dot(pooled, w1_ref[...], preferred_element_type=jnp.float32) + b1_ref[...]
    z_ref[...] = z

    @pl.when(pl.program_id(0) == 0)
    def _init():
        s_ref[...] = jnp.zeros_like(s_ref)
        ss_ref[...] = jnp.zeros_like(ss_ref)

    s_ref[...] += jnp.sum(z, axis=0, keepdims=True)
    ss_ref[...] += jnp.sum(z * z, axis=0, keepdims=True)


def _pass2_kernel(z_ref, s_ref, ss_ref, w2_ref, b2_ref, g1_ref, be1_ref,
                  r_ref, s2_ref, ss2_ref):
    m = s_ref[...] / N
    v = ss_ref[...] / N - m * m
    scale = g1_ref[...] * jax.lax.rsqrt(v + EPS)
    zn = jax.nn.relu((z_ref[...] - m) * scale + be1_ref[...])
    r = jnp.dot(zn, w2_ref[...], preferred_element_type=jnp.float32) + b2_ref[...]
    r_ref[...] = r

    @pl.when(pl.program_id(0) == 0)
    def _init():
        s2_ref[...] = jnp.zeros_like(s2_ref)
        ss2_ref[...] = jnp.zeros_like(ss2_ref)

    s2_ref[...] += jnp.sum(r, axis=0, keepdims=True)
    ss2_ref[...] += jnp.sum(r * r, axis=0, keepdims=True)


def _pass3_kernel(r_ref, s2_ref, ss2_ref, g_ref, be_ref, gpt_ref,
                  h_ref, ph_ref):
    m = s2_ref[...] / N
    v = ss2_ref[...] / N - m * m
    scale = g_ref[...] * jax.lax.rsqrt(v + EPS)
    h = jax.nn.relu((r_ref[...] - m) * scale + be_ref[...])
    h_ref[...] = h

    @pl.when(pl.program_id(0) == 0)
    def _init():
        ph_ref[...] = jnp.zeros_like(ph_ref)

    ph_ref[...] += jax.lax.dot_general(
        gpt_ref[...], h, (((0,), (0,)), ((), ())),
        preferred_element_type=jnp.float32)


@functools.partial(jax.jit, static_argnames=("interpret",))
def kernel(x, graph_pool, padded_nei, adj, W1_0, b1_0, W2_0, b2_0,
           g1_0, be1_0, g_0, be_0, interpret=False):
    del padded_nei
    b1 = b1_0.reshape(1, H)
    b2 = b2_0.reshape(1, H)
    g1 = g1_0.reshape(1, H)
    be1 = be1_0.reshape(1, H)
    g = g_0.reshape(1, H)
    be = be_0.reshape(1, H)

    n1 = N // TM1
    z, s1, ss1 = pl.pallas_call(
        _pass1_kernel,
        grid=(n1,),
        in_specs=[
            pl.BlockSpec((N, D), lambda i: (0, 0)),
            pl.BlockSpec((TM1, N), lambda i: (i, 0)),
            pl.BlockSpec((D, H), lambda i: (0, 0)),
            pl.BlockSpec((1, H), lambda i: (0, 0)),
        ],
        out_specs=[
            pl.BlockSpec((TM1, H), lambda i: (i, 0)),
            pl.BlockSpec((1, H), lambda i: (0, 0)),
            pl.BlockSpec((1, H), lambda i: (0, 0)),
        ],
        out_shape=[
            jax.ShapeDtypeStruct((N, H), jnp.float32),
            jax.ShapeDtypeStruct((1, H), jnp.float32),
            jax.ShapeDtypeStruct((1, H), jnp.float32),
        ],
        interpret=interpret,
    )(x, adj, W1_0, b1)

    n2 = N // TM2
    r, s2, ss2 = pl.pallas_call(
        _pass2_kernel,
        grid=(n2,),
        in_specs=[
            pl.BlockSpec((TM2, H), lambda i: (i, 0)),
            pl.BlockSpec((1, H), lambda i: (0, 0)),
            pl.BlockSpec((1, H), lambda i: (0, 0)),
            pl.BlockSpec((H, H), lambda i: (0, 0)),
            pl.BlockSpec((1, H), lambda i: (0, 0)),
            pl.BlockSpec((1, H), lambda i: (0, 0)),
            pl.BlockSpec((1, H), lambda i: (0, 0)),
        ],
        out_specs=[
            pl.BlockSpec((TM2, H), lambda i: (i, 0)),
            pl.BlockSpec((1, H), lambda i: (0, 0)),
            pl.BlockSpec((1, H), lambda i: (0, 0)),
        ],
        out_shape=[
            jax.ShapeDtypeStruct((N, H), jnp.float32),
            jax.ShapeDtypeStruct((1, H), jnp.float32),
            jax.ShapeDtypeStruct((1, H), jnp.float32),
        ],
        interpret=interpret,
    )(z, s1, ss1, W2_0, b2, g1, be1)

    h_nodes, pooled_h = pl.pallas_call(
        _pass3_kernel,
        grid=(n2,),
        in_specs=[
            pl.BlockSpec((TM2, H), lambda i: (i, 0)),
            pl.BlockSpec((1, H), lambda i: (0, 0)),
            pl.BlockSpec((1, H), lambda i: (0, 0)),
            pl.BlockSpec((1, H), lambda i: (0, 0)),
            pl.BlockSpec((1, H), lambda i: (0, 0)),
            pl.BlockSpec((TM2, G), lambda i: (i, 0)),
        ],
        out_specs=[
            pl.BlockSpec((TM2, H), lambda i: (i, 0)),
            pl.BlockSpec((G, H), lambda i: (0, 0)),
        ],
        out_shape=[
            jax.ShapeDtypeStruct((N, H), jnp.float32),
            jax.ShapeDtypeStruct((G, H), jnp.float32),
        ],
        interpret=interpret,
    )(r, s2, ss2, g, be, graph_pool.T)

    return (pooled_h, h_nodes)


# speedup vs baseline: 1.0090x; 1.0090x over previous
import functools

import jax
import jax.numpy as jnp
from jax.experimental import pallas as pl

N = 10000
D = 128
H = 128
G = 64
EPS = 1e-5

TM1 = 400
TM2 = 1000


def _pass1_kernel(x_ref, adj_ref, w1_ref, b1_ref, z_ref, s_ref, ss_ref):
    pooled = jnp.dot(adj_ref[...].astype(jnp.bfloat16),
                     x_ref[...].astype(jnp.bfloat16),
                     preferred_element_type=jnp.float32)
    z = jnp.dot(pooled, w1_ref[...], preferred_element_type=jnp.float32) + b1_ref[...]
    z_ref[...] = z

    @pl.when(pl.program_id(0) == 0)
    def _init():
        s_ref[...] = jnp.zeros_like(s_ref)
        ss_ref[...] = jnp.zeros_like(ss_ref)

    s_ref[...] += jnp.sum(z, axis=0, keepdims=True)
    ss_ref[...] += jnp.sum(z * z, axis=0, keepdims=True)


def _pass2_kernel(z_ref, s_ref, ss_ref, w2_ref, b2_ref, g1_ref, be1_ref,
                  r_ref, s2_ref, ss2_ref):
    m = s_ref[...] / N
    v = ss_ref[...] / N - m * m
    scale = g1_ref[...] * jax.lax.rsqrt(v + EPS)
    zn = jax.nn.relu((z_ref[...] - m) * scale + be1_ref[...])
    r = jnp.dot(zn, w2_ref[...], preferred_element_type=jnp.float32) + b2_ref[...]
    r_ref[...] = r

    @pl.when(pl.program_id(0) == 0)
    def _init():
        s2_ref[...] = jnp.zeros_like(s2_ref)
        ss2_ref[...] = jnp.zeros_like(ss2_ref)

    s2_ref[...] += jnp.sum(r, axis=0, keepdims=True)
    ss2_ref[...] += jnp.sum(r * r, axis=0, keepdims=True)


def _pass3_kernel(r_ref, s2_ref, ss2_ref, g_ref, be_ref, gpt_ref,
                  h_ref, ph_ref):
    m = s2_ref[...] / N
    v = ss2_ref[...] / N - m * m
    scale = g_ref[...] * jax.lax.rsqrt(v + EPS)
    h = jax.nn.relu((r_ref[...] - m) * scale + be_ref[...])
    h_ref[...] = h

    @pl.when(pl.program_id(0) == 0)
    def _init():
        ph_ref[...] = jnp.zeros_like(ph_ref)

    ph_ref[...] += jax.lax.dot_general(
        gpt_ref[...], h, (((0,), (0,)), ((), ())),
        preferred_element_type=jnp.float32)


@functools.partial(jax.jit, static_argnames=("interpret",))
def kernel(x, graph_pool, padded_nei, adj, W1_0, b1_0, W2_0, b2_0,
           g1_0, be1_0, g_0, be_0, interpret=False):
    del padded_nei
    b1 = b1_0.reshape(1, H)
    b2 = b2_0.reshape(1, H)
    g1 = g1_0.reshape(1, H)
    be1 = be1_0.reshape(1, H)
    g = g_0.reshape(1, H)
    be = be_0.reshape(1, H)

    n1 = N // TM1
    z, s1, ss1 = pl.pallas_call(
        _pass1_kernel,
        grid=(n1,),
        in_specs=[
            pl.BlockSpec((N, D), lambda i: (0, 0)),
            pl.BlockSpec((TM1, N), lambda i: (i, 0)),
            pl.BlockSpec((D, H), lambda i: (0, 0)),
            pl.BlockSpec((1, H), lambda i: (0, 0)),
        ],
        out_specs=[
            pl.BlockSpec((TM1, H), lambda i: (i, 0)),
            pl.BlockSpec((1, H), lambda i: (0, 0)),
            pl.BlockSpec((1, H), lambda i: (0, 0)),
        ],
        out_shape=[
            jax.ShapeDtypeStruct((N, H), jnp.float32),
            jax.ShapeDtypeStruct((1, H), jnp.float32),
            jax.ShapeDtypeStruct((1, H), jnp.float32),
        ],
        interpret=interpret,
    )(x, adj, W1_0, b1)

    n2 = N // TM2
    r, s2, ss2 = pl.pallas_call(
        _pass2_kernel,
        grid=(n2,),
        in_specs=[
            pl.BlockSpec((TM2, H), lambda i: (i, 0)),
            pl.BlockSpec((1, H), lambda i: (0, 0)),
            pl.BlockSpec((1, H), lambda i: (0, 0)),
            pl.BlockSpec((H, H), lambda i: (0, 0)),
            pl.BlockSpec((1, H), lambda i: (0, 0)),
            pl.BlockSpec((1, H), lambda i: (0, 0)),
            pl.BlockSpec((1, H), lambda i: (0, 0)),
        ],
        out_specs=[
            pl.BlockSpec((TM2, H), lambda i: (i, 0)),
            pl.BlockSpec((1, H), lambda i: (0, 0)),
            pl.BlockSpec((1, H), lambda i: (0, 0)),
        ],
        out_shape=[
            jax.ShapeDtypeStruct((N, H), jnp.float32),
            jax.ShapeDtypeStruct((1, H), jnp.float32),
            jax.ShapeDtypeStruct((1, H), jnp.float32),
        ],
        interpret=interpret,
    )(z, s1, ss1, W2_0, b2, g1, be1)

    h_nodes, pooled_h = pl.pallas_call(
        _pass3_kernel,
        grid=(n2,),
        in_specs=[
            pl.BlockSpec((TM2, H), lambda i: (i, 0)),
            pl.BlockSpec((1, H), lambda i: (0, 0)),
            pl.BlockSpec((1, H), lambda i: (0, 0)),
            pl.BlockSpec((1, H), lambda i: (0, 0)),
            pl.BlockSpec((1, H), lambda i: (0, 0)),
            pl.BlockSpec((TM2, G), lambda i: (i, 0)),
        ],
        out_specs=[
            pl.BlockSpec((TM2, H), lambda i: (i, 0)),
            pl.BlockSpec((G, H), lambda i: (0, 0)),
        ],
        out_shape=[
            jax.ShapeDtypeStruct((N, H), jnp.float32),
            jax.ShapeDtypeStruct((G, H), jnp.float32),
        ],
        interpret=interpret,
    )(r, s2, ss2, g, be, graph_pool.T)

    return (pooled_h, h_nodes)
